# baseline (speedup 1.0000x reference)
import jax
import jax.numpy as jnp
import numpy as np
from jax import lax
from jax.experimental import pallas as pl
from jax.experimental.pallas import tpu as pltpu
from jax.experimental.pallas import tpu_sc as plsc

_VOCAB = 32000
_MAX_LEN = 2048
_D = 2048
_NGRP = _D // 32

_NC = 2
_NS = 16
_NW = _NC * _NS

_BATCH = 4
_POS_PER_W = _MAX_LEN // _NW
_CHUNK = 8
_NPB = _POS_PER_W // _CHUNK
_NITEM = _NPB * _BATCH
_NBUF = 4


def _positional_encoding_np(max_len, d_model):
    pos = np.arange(max_len, dtype=np.float64)[:, None]
    j = np.arange(d_model // 2, dtype=np.float64)[None, :]
    angle = pos / np.power(10000.0, 2.0 * j / d_model)
    pe = np.zeros((max_len, d_model), dtype=np.float32)
    pe[:, 0::2] = np.sin(angle)
    pe[:, 1::2] = np.cos(angle)
    return pe


def _pe_bf16_interleaved(pe):
    L, D = pe.shape
    x = pe.reshape(L, D // 32, 2, 16).transpose(0, 1, 3, 2).reshape(L * D)
    xb = jnp.asarray(x).astype(jnp.bfloat16).reshape(L * D // 2, 2)
    return lax.bitcast_convert_type(xb, jnp.int32)


_PE = _positional_encoding_np(_MAX_LEN, _D)


def _emb_pe_body(x_hbm, pe_hbm, table_hbm, out_hbm,
                 idx_v, rows0, rows1, rows2, rows3, pe0, pe1,
                 gsem0, gsem1, gsem2, gsem3,
                 ssem0, ssem1, ssem2, ssem3, psem):
    wid = lax.axis_index("s") * _NC + lax.axis_index("c")
    pos0 = wid * _POS_PER_W

    rows = (rows0, rows1, rows2, rows3)
    gsem = (gsem0, gsem1, gsem2, gsem3)
    ssem = (ssem0, ssem1, ssem2, ssem3)
    pe = (pe0, pe1)

    def item_pb_b(k):
        return divmod(k, _BATCH)

    def item_off(k):
        pb, b = item_pb_b(k)
        return b * _MAX_LEN + pos0 + pb * _CHUNK

    def gather(k):
        pb, b = item_pb_b(k)
        buf = k % _NBUF
        return pltpu.async_copy(
            table_hbm.at[idx_v.at[b, pl.ds(pb * _CHUNK, _CHUNK)]],
            rows[buf], gsem[buf])

    def stage_pe(pb):
        return pltpu.async_copy(
            pe_hbm.at[pl.ds((pos0 + pb * _CHUNK) * (_D // 2),
                            _CHUNK * _D // 2)],
            pe[pb % 2], psem)

    def add_pe(rbuf, pbuf):
        @plsc.parallel_loop(0, _CHUNK * _NGRP, unroll=8)
        def add(i):
            r = i // _NGRP
            col = (i % _NGRP) * 32
            w = pbuf[pl.ds(i * 16, 16)]
            a = lax.bitcast_convert_type(lax.shift_left(w, 16),
                                         jnp.float32)
            b = lax.bitcast_convert_type(
                lax.bitwise_and(w, jnp.int32(-65536)), jnp.float32)
            plsc.addupdate(rbuf.at[r, pl.ds(col, 16)], a)
            plsc.addupdate(rbuf.at[r, pl.ds(col + 16, 16)], b)

    for b in range(_BATCH):
        pltpu.sync_copy(x_hbm.at[pl.ds(b * _MAX_LEN + pos0, _POS_PER_W)],
                        idx_v.at[b])
    pe_d = [stage_pe(0), stage_pe(1)]
    gd = [None] * _NBUF
    sd = [None] * _NBUF
    for k in range(_NBUF - 1):
        gd[k] = gather(k)

    for k in range(_NITEM):
        pb, b = item_pb_b(k)
        buf = k % _NBUF
        if b == 0 and pe_d[pb % 2] is not None:
            pe_d[pb % 2].wait()
            pe_d[pb % 2] = None
        gd[buf].wait()
        add_pe(rows[buf], pe[pb % 2])
        if b == _BATCH - 1 and pb + 2 < _NPB:
            pe_d[pb % 2] = stage_pe(pb + 2)
        sd[buf] = pltpu.async_copy(
            rows[buf], out_hbm.at[pl.ds(item_off(k), _CHUNK)], ssem[buf])
        nk = k + _NBUF - 1
        if nk < _NITEM:
            nbuf = nk % _NBUF
            if sd[nbuf] is not None:
                sd[nbuf].wait()
            gd[nbuf] = gather(nk)

    for d in sd:
        if d is not None:
            d.wait()


@jax.jit
def _emb_pe(x_flat, pe_b, W_emb):
    mesh = plsc.VectorSubcoreMesh(core_axis_name="c", subcore_axis_name="s")
    return pl.kernel(
        _emb_pe_body,
        out_type=jax.ShapeDtypeStruct((_BATCH * _MAX_LEN, _D), jnp.float32),
        mesh=mesh,
        scratch_types=[
            pltpu.VMEM((_BATCH, _POS_PER_W), jnp.int32),
            pltpu.VMEM((_CHUNK, _D), jnp.float32),
            pltpu.VMEM((_CHUNK, _D), jnp.float32),
            pltpu.VMEM((_CHUNK, _D), jnp.float32),
            pltpu.VMEM((_CHUNK, _D), jnp.float32),
            pltpu.VMEM((_CHUNK * _D // 2,), jnp.int32),
            pltpu.VMEM((_CHUNK * _D // 2,), jnp.int32),
            pltpu.SemaphoreType.DMA,
            pltpu.SemaphoreType.DMA,
            pltpu.SemaphoreType.DMA,
            pltpu.SemaphoreType.DMA,
            pltpu.SemaphoreType.DMA,
            pltpu.SemaphoreType.DMA,
            pltpu.SemaphoreType.DMA,
            pltpu.SemaphoreType.DMA,
            pltpu.SemaphoreType.DMA,
        ],
    )(x_flat, pe_b, W_emb)


def kernel(x, W_emb):
    b, s = x.shape
    x_flat = x.reshape(-1).astype(jnp.int32)
    out = _emb_pe(x_flat, _pe_bf16_interleaved(_PE), W_emb)
    return out.reshape(b, s, _D)


# speedup vs baseline: 1.9187x; 1.9187x over previous
import jax
import jax.numpy as jnp
import numpy as np
from jax import lax
from jax.experimental import pallas as pl
from jax.experimental.pallas import tpu as pltpu
from jax.experimental.pallas import tpu_sc as plsc

_VOCAB = 32000
_MAX_LEN = 2048
_D = 2048
_NVEC = _D // 16

_NC = 2
_NS = 16
_NW = _NC * _NS

_BATCH = 4
_POS_PER_W = _MAX_LEN // _NW
_CHUNK = 8
_NPB = _POS_PER_W // _CHUNK
_NITEM = _NPB * _BATCH
_NBUF = 4


def _positional_encoding_np(max_len, d_model):
    pos = np.arange(max_len, dtype=np.float64)[:, None]
    j = np.arange(d_model // 2, dtype=np.float64)[None, :]
    angle = pos / np.power(10000.0, 2.0 * j / d_model)
    pe = np.zeros((max_len, d_model), dtype=np.float32)
    pe[:, 0::2] = np.sin(angle)
    pe[:, 1::2] = np.cos(angle)
    return pe


_PE = _positional_encoding_np(_MAX_LEN, _D)


def _emb_pe_body(x_hbm, pe_hbm, table_hbm, out_hbm,
                 idx_v, rows0, rows1, rows2, rows3, pe0, pe1,
                 gsem0, gsem1, gsem2, gsem3,
                 ssem0, ssem1, ssem2, ssem3, psem):
    wid = lax.axis_index("s") * _NC + lax.axis_index("c")
    pos0 = wid * _POS_PER_W

    rows = (rows0, rows1, rows2, rows3)
    gsem = (gsem0, gsem1, gsem2, gsem3)
    ssem = (ssem0, ssem1, ssem2, ssem3)
    pe = (pe0, pe1)

    def item_pb_b(k):
        return divmod(k, _BATCH)

    def item_off(k):
        pb, b = item_pb_b(k)
        return b * _MAX_LEN + pos0 + pb * _CHUNK

    def gather(k):
        pb, b = item_pb_b(k)
        buf = k % _NBUF
        return pltpu.async_copy(
            table_hbm.at[idx_v.at[b, pl.ds(pb * _CHUNK, _CHUNK)]],
            rows[buf], gsem[buf])

    def stage_pe(pb):
        return pltpu.async_copy(
            pe_hbm.at[pl.ds(pos0 + pb * _CHUNK, _CHUNK)], pe[pb % 2], psem)

    def add_pe(rbuf, pbuf):
        @plsc.parallel_loop(0, _CHUNK * _NVEC, unroll=8)
        def add(i):
            r = i // _NVEC
            col = (i % _NVEC) * 16
            plsc.addupdate(rbuf.at[r, pl.ds(col, 16)],
                           pbuf[r, pl.ds(col, 16)])

    for b in range(_BATCH):
        pltpu.sync_copy(x_hbm.at[pl.ds(b * _MAX_LEN + pos0, _POS_PER_W)],
                        idx_v.at[b])
    pe_d = [stage_pe(0), stage_pe(1)]
    gd = [None] * _NBUF
    sd = [None] * _NBUF
    for k in range(_NBUF - 1):
        gd[k] = gather(k)

    for k in range(_NITEM):
        pb, b = item_pb_b(k)
        buf = k % _NBUF
        if b == 0 and pe_d[pb % 2] is not None:
            pe_d[pb % 2].wait()
            pe_d[pb % 2] = None
        gd[buf].wait()
        add_pe(rows[buf], pe[pb % 2])
        if b == _BATCH - 1 and pb + 2 < _NPB:
            pe_d[pb % 2] = stage_pe(pb + 2)
        sd[buf] = pltpu.async_copy(
            rows[buf], out_hbm.at[pl.ds(item_off(k), _CHUNK)], ssem[buf])
        nk = k + _NBUF - 1
        if nk < _NITEM:
            nbuf = nk % _NBUF
            if sd[nbuf] is not None:
                sd[nbuf].wait()
            gd[nbuf] = gather(nk)

    for d in sd:
        if d is not None:
            d.wait()


@jax.jit
def _emb_pe(x_flat, pe_b, W_emb):
    mesh = plsc.VectorSubcoreMesh(core_axis_name="c", subcore_axis_name="s")
    return pl.kernel(
        _emb_pe_body,
        out_type=jax.ShapeDtypeStruct((_BATCH * _MAX_LEN, _D), jnp.float32),
        mesh=mesh,
        scratch_types=[
            pltpu.VMEM((_BATCH, _POS_PER_W), jnp.int32),
            pltpu.VMEM((_CHUNK, _D), jnp.float32),
            pltpu.VMEM((_CHUNK, _D), jnp.float32),
            pltpu.VMEM((_CHUNK, _D), jnp.float32),
            pltpu.VMEM((_CHUNK, _D), jnp.float32),
            pltpu.VMEM((_CHUNK, _D), jnp.float32),
            pltpu.VMEM((_CHUNK, _D), jnp.float32),
            pltpu.SemaphoreType.DMA,
            pltpu.SemaphoreType.DMA,
            pltpu.SemaphoreType.DMA,
            pltpu.SemaphoreType.DMA,
            pltpu.SemaphoreType.DMA,
            pltpu.SemaphoreType.DMA,
            pltpu.SemaphoreType.DMA,
            pltpu.SemaphoreType.DMA,
            pltpu.SemaphoreType.DMA,
        ],
    )(x_flat, pe_b, W_emb)


def kernel(x, W_emb):
    b, s = x.shape
    x_flat = x.reshape(-1).astype(jnp.int32)
    out = _emb_pe(x_flat, jnp.asarray(_PE), W_emb)
    return out.reshape(b, s, _D)


# speedup vs baseline: 1.9446x; 1.0135x over previous
import jax
import jax.numpy as jnp
import numpy as np
from jax import lax
from jax.experimental import pallas as pl
from jax.experimental.pallas import tpu as pltpu
from jax.experimental.pallas import tpu_sc as plsc

_VOCAB = 32000
_MAX_LEN = 2048
_D = 2048
_NVEC = _D // 16

_NC = 2
_NS = 16
_NW = _NC * _NS

_BATCH = 4
_POS_PER_W = _MAX_LEN // _NW
_CHUNK = 8
_NPB = _POS_PER_W // _CHUNK
_NITEM = _NPB * _BATCH
_NBUF = 4


def _positional_encoding_np(max_len, d_model):
    pos = np.arange(max_len, dtype=np.float64)[:, None]
    j = np.arange(d_model // 2, dtype=np.float64)[None, :]
    angle = pos / np.power(10000.0, 2.0 * j / d_model)
    pe = np.zeros((max_len, d_model), dtype=np.float32)
    pe[:, 0::2] = np.sin(angle)
    pe[:, 1::2] = np.cos(angle)
    return pe


_PE = _positional_encoding_np(_MAX_LEN, _D)


def _emb_pe_body(x_hbm, pe_hbm, table_hbm, out_hbm,
                 idx_v, rows0, rows1, rows2, rows3, pe0, pe1,
                 gsem0, gsem1, gsem2, gsem3,
                 ssem0, ssem1, ssem2, ssem3, psem):
    wid = lax.axis_index("s") * _NC + lax.axis_index("c")
    pos0 = wid * _POS_PER_W

    rows = (rows0, rows1, rows2, rows3)
    gsem = (gsem0, gsem1, gsem2, gsem3)
    ssem = (ssem0, ssem1, ssem2, ssem3)
    pe = (pe0, pe1)

    def item_pb_b(k):
        return divmod(k, _BATCH)

    def item_off(k):
        pb, b = item_pb_b(k)
        return b * _MAX_LEN + pos0 + pb * _CHUNK

    def gather(k):
        pb, b = item_pb_b(k)
        buf = k % _NBUF
        return pltpu.async_copy(
            table_hbm.at[idx_v.at[b, pl.ds(pb * _CHUNK, _CHUNK)]],
            rows[buf], gsem[buf])

    def stage_pe(pb):
        return pltpu.async_copy(
            pe_hbm.at[pl.ds(pos0 + pb * _CHUNK, _CHUNK)], pe[pb % 2], psem)

    def add_pe(rbuf, pbuf):
        @plsc.parallel_loop(0, _CHUNK * _NVEC, unroll=8)
        def add(i):
            r = i // _NVEC
            col = (i % _NVEC) * 16
            plsc.addupdate(rbuf.at[r, pl.ds(col, 16)],
                           pbuf[r, pl.ds(col, 16)])

    idx_d = [
        pltpu.async_copy(x_hbm.at[pl.ds(b * _MAX_LEN + pos0, _POS_PER_W)],
                         idx_v.at[b], psem)
        for b in range(_BATCH)
    ]
    pe_d = [stage_pe(0), stage_pe(1)]
    for d in idx_d:
        d.wait()
    gd = [None] * _NBUF
    sd = [None] * _NBUF
    for k in range(_NBUF - 1):
        gd[k] = gather(k)

    for k in range(_NITEM):
        pb, b = item_pb_b(k)
        buf = k % _NBUF
        if b == 0 and pe_d[pb % 2] is not None:
            pe_d[pb % 2].wait()
            pe_d[pb % 2] = None
        gd[buf].wait()
        add_pe(rows[buf], pe[pb % 2])
        if b == _BATCH - 1 and pb + 2 < _NPB:
            pe_d[pb % 2] = stage_pe(pb + 2)
        sd[buf] = pltpu.async_copy(
            rows[buf], out_hbm.at[pl.ds(item_off(k), _CHUNK)], ssem[buf])
        nk = k + _NBUF - 1
        if nk < _NITEM:
            nbuf = nk % _NBUF
            if sd[nbuf] is not None:
                sd[nbuf].wait()
            gd[nbuf] = gather(nk)

    for d in sd:
        if d is not None:
            d.wait()


@jax.jit
def _emb_pe(x_flat, pe_b, W_emb):
    mesh = plsc.VectorSubcoreMesh(core_axis_name="c", subcore_axis_name="s")
    return pl.kernel(
        _emb_pe_body,
        out_type=jax.ShapeDtypeStruct((_BATCH * _MAX_LEN, _D), jnp.float32),
        mesh=mesh,
        scratch_types=[
            pltpu.VMEM((_BATCH, _POS_PER_W), jnp.int32),
            pltpu.VMEM((_CHUNK, _D), jnp.float32),
            pltpu.VMEM((_CHUNK, _D), jnp.float32),
            pltpu.VMEM((_CHUNK, _D), jnp.float32),
            pltpu.VMEM((_CHUNK, _D), jnp.float32),
            pltpu.VMEM((_CHUNK, _D), jnp.float32),
            pltpu.VMEM((_CHUNK, _D), jnp.float32),
            pltpu.SemaphoreType.DMA,
            pltpu.SemaphoreType.DMA,
            pltpu.SemaphoreType.DMA,
            pltpu.SemaphoreType.DMA,
            pltpu.SemaphoreType.DMA,
            pltpu.SemaphoreType.DMA,
            pltpu.SemaphoreType.DMA,
            pltpu.SemaphoreType.DMA,
            pltpu.SemaphoreType.DMA,
        ],
    )(x_flat, pe_b, W_emb)


def kernel(x, W_emb):
    b, s = x.shape
    x_flat = x.reshape(-1).astype(jnp.int32)
    out = _emb_pe(x_flat, jnp.asarray(_PE), W_emb)
    return out.reshape(b, s, _D)


# speedup vs baseline: 1.9447x; 1.0000x over previous
import jax
import jax.numpy as jnp
import numpy as np
from jax import lax
from jax.experimental import pallas as pl
from jax.experimental.pallas import tpu as pltpu
from jax.experimental.pallas import tpu_sc as plsc

_VOCAB = 32000
_MAX_LEN = 2048
_D = 2048
_NVEC = _D // 16

_NC = 2
_NS = 16
_NW = _NC * _NS

_BATCH = 4
_POS_PER_W = _MAX_LEN // _NW
_CHUNK = 8
_NPB = _POS_PER_W // _CHUNK
_NITEM = _NPB * _BATCH
_NBUF = 4


def _positional_encoding_np(max_len, d_model):
    pos = np.arange(max_len, dtype=np.float64)[:, None]
    j = np.arange(d_model // 2, dtype=np.float64)[None, :]
    angle = pos / np.power(10000.0, 2.0 * j / d_model)
    pe = np.zeros((max_len, d_model), dtype=np.float32)
    pe[:, 0::2] = np.sin(angle)
    pe[:, 1::2] = np.cos(angle)
    return pe


_PE = _positional_encoding_np(_MAX_LEN, _D)


def _emb_pe_body(x_hbm, pe_hbm, table_hbm, out_hbm,
                 idx_v, rows0, rows1, rows2, rows3, pe0, pe1,
                 gsem0, gsem1, gsem2, gsem3,
                 ssem0, ssem1, ssem2, ssem3, psem, isem):
    wid = lax.axis_index("s") * _NC + lax.axis_index("c")
    pos0 = wid * _POS_PER_W

    rows = (rows0, rows1, rows2, rows3)
    gsem = (gsem0, gsem1, gsem2, gsem3)
    ssem = (ssem0, ssem1, ssem2, ssem3)
    pe = (pe0, pe1)

    def item_pb_b(k):
        return divmod(k, _BATCH)

    def item_off(k):
        pb, b = item_pb_b(k)
        return b * _MAX_LEN + pos0 + pb * _CHUNK

    def gather(k):
        pb, b = item_pb_b(k)
        buf = k % _NBUF
        return pltpu.async_copy(
            table_hbm.at[idx_v.at[b, pl.ds(pb * _CHUNK, _CHUNK)]],
            rows[buf], gsem[buf])

    def stage_pe(pb):
        return pltpu.async_copy(
            pe_hbm.at[pl.ds(pos0 + pb * _CHUNK, _CHUNK)], pe[pb % 2], psem)

    def add_pe(rbuf, pbuf):
        @plsc.parallel_loop(0, _CHUNK * _NVEC, unroll=8)
        def add(i):
            r = i // _NVEC
            col = (i % _NVEC) * 16
            plsc.addupdate(rbuf.at[r, pl.ds(col, 16)],
                           pbuf[r, pl.ds(col, 16)])

    idx_d = [
        pltpu.async_copy(x_hbm.at[pl.ds(b * _MAX_LEN + pos0, _POS_PER_W)],
                         idx_v.at[b], isem)
        for b in range(_BATCH)
    ]
    pe_d = [stage_pe(0), stage_pe(1)]
    for d in idx_d:
        d.wait()
    gd = [None] * _NBUF
    sd = [None] * _NBUF
    for k in range(_NBUF - 1):
        gd[k] = gather(k)

    for k in range(_NITEM):
        pb, b = item_pb_b(k)
        buf = k % _NBUF
        if b == 0 and pe_d[pb % 2] is not None:
            pe_d[pb % 2].wait()
            pe_d[pb % 2] = None
        gd[buf].wait()
        add_pe(rows[buf], pe[pb % 2])
        if b == _BATCH - 1 and pb + 2 < _NPB:
            pe_d[pb % 2] = stage_pe(pb + 2)
        sd[buf] = pltpu.async_copy(
            rows[buf], out_hbm.at[pl.ds(item_off(k), _CHUNK)], ssem[buf])
        nk = k + _NBUF - 1
        if nk < _NITEM:
            nbuf = nk % _NBUF
            if sd[nbuf] is not None:
                sd[nbuf].wait()
            gd[nbuf] = gather(nk)

    for d in sd:
        if d is not None:
            d.wait()


@jax.jit
def _emb_pe(x_flat, pe_b, W_emb):
    mesh = plsc.VectorSubcoreMesh(core_axis_name="c", subcore_axis_name="s")
    return pl.kernel(
        _emb_pe_body,
        out_type=jax.ShapeDtypeStruct((_BATCH * _MAX_LEN, _D), jnp.float32),
        mesh=mesh,
        scratch_types=[
            pltpu.VMEM((_BATCH, _POS_PER_W), jnp.int32),
            pltpu.VMEM((_CHUNK, _D), jnp.float32),
            pltpu.VMEM((_CHUNK, _D), jnp.float32),
            pltpu.VMEM((_CHUNK, _D), jnp.float32),
            pltpu.VMEM((_CHUNK, _D), jnp.float32),
            pltpu.VMEM((_CHUNK, _D), jnp.float32),
            pltpu.VMEM((_CHUNK, _D), jnp.float32),
            pltpu.SemaphoreType.DMA,
            pltpu.SemaphoreType.DMA,
            pltpu.SemaphoreType.DMA,
            pltpu.SemaphoreType.DMA,
            pltpu.SemaphoreType.DMA,
            pltpu.SemaphoreType.DMA,
            pltpu.SemaphoreType.DMA,
            pltpu.SemaphoreType.DMA,
            pltpu.SemaphoreType.DMA,
            pltpu.SemaphoreType.DMA,
        ],
    )(x_flat, pe_b, W_emb)


def kernel(x, W_emb):
    b, s = x.shape
    x_flat = x.reshape(-1).astype(jnp.int32)
    out = _emb_pe(x_flat, jnp.asarray(_PE), W_emb)
    return out.reshape(b, s, _D)
